# baseline (speedup 1.0000x reference)
import functools

import jax
import jax.numpy as jnp
from jax import lax
from jax.experimental import pallas as pl
from jax.experimental.pallas import tpu as pltpu
from jax.experimental.pallas import tpu_sc as plsc

_BATCH = 16384
_ED = 32
_LANES = 128
_PACK = _LANES // _ED

_VQ_BIG = 25088
_VQ_SMALL = 256
_PBLK = 512

_NC, _NS = 2, 16
_NW = _NC * _NS
_BPW = _BATCH // _NW
_HALF = _BPW // 2
_CHUNK = 128

_MLP_BLOCK = 4096


def _pack2_body(a_ref, b_ref, oa, ob):
    s = pl.program_id(1)
    for sv in range(_PACK):
        @pl.when(s == sv)
        def _():
            oa[:, _ED * sv:_ED * (sv + 1)] = a_ref[...]
            ob[:, _ED * sv:_ED * (sv + 1)] = b_ref[...]


def _pack2(ta, tb, vq, blk):
    nblk = vq // blk
    in_spec = pl.BlockSpec((blk, _ED), lambda i, s: (s * nblk + i, 0))
    out_spec = pl.BlockSpec((blk, _LANES), lambda i, s: (i, 0))
    return pl.pallas_call(
        _pack2_body,
        grid=(nblk, _PACK),
        in_specs=[in_spec, in_spec],
        out_specs=[out_spec, out_spec],
        out_shape=[jax.ShapeDtypeStruct((vq, _LANES), jnp.float32)] * 2,
    )(ta, tb)


def _gather4(p0, p1, p2, p3, t0, t1, t2, t3):
    mesh = plsc.VectorSubcoreMesh(core_axis_name="c", subcore_axis_name="s")
    out = jax.ShapeDtypeStruct((_BATCH, _LANES), jnp.float32)
    fp = jnp.float32

    @functools.partial(
        pl.kernel, out_type=(out, out, out, out), mesh=mesh,
        scratch_types=[
            pltpu.VMEM((_BPW,), jnp.int32), pltpu.VMEM((_BPW,), jnp.int32),
            pltpu.VMEM((_BPW,), jnp.int32), pltpu.VMEM((_BPW,), jnp.int32),
            pltpu.VMEM((_HALF, _LANES), fp), pltpu.VMEM((_HALF, _LANES), fp),
            pltpu.SemaphoreType.DMA, pltpu.SemaphoreType.DMA,
            pltpu.SemaphoreType.DMA, pltpu.SemaphoreType.DMA,
        ])
    def gather_kernel(i0_hbm, i1_hbm, i2_hbm, i3_hbm,
                      t0_hbm, t1_hbm, t2_hbm, t3_hbm,
                      o0_hbm, o1_hbm, o2_hbm, o3_hbm,
                      iv0, iv1, iv2, iv3, rows0, rows1,
                      sg0, sg1, sw0, sw1):
        wid = lax.axis_index("s") * _NC + lax.axis_index("c")
        base = wid * _BPW
        i_hbms = (i0_hbm, i1_hbm, i2_hbm, i3_hbm)
        t_hbms = (t0_hbm, t1_hbm, t2_hbm, t3_hbm)
        o_hbms = (o0_hbm, o1_hbm, o2_hbm, o3_hbm)
        ivs = (iv0, iv1, iv2, iv3)
        rows = (rows0, rows1)
        sgs = (sg0, sg1)
        sws = (sw0, sw1)
        for k in range(4):
            pltpu.sync_copy(i_hbms[k].at[pl.ds(base, _BPW)], ivs[k])
        wdescs = []
        items = [(k, h) for k in range(4) for h in range(2)]
        for i, (k, h) in enumerate(items):
            b = i % 2
            if i >= 2:
                wdescs[i - 2].wait()
            gd = []
            for c in range(_HALF // _CHUNK):
                isl = pl.ds(h * _HALF + c * _CHUNK, _CHUNK)
                gd.append(pltpu.async_copy(
                    t_hbms[k].at[ivs[k].at[isl]],
                    rows[b].at[pl.ds(c * _CHUNK, _CHUNK)], sgs[b]))
            for d in gd:
                d.wait()
            wdescs.append(pltpu.async_copy(
                rows[b], o_hbms[k].at[pl.ds(base + h * _HALF, _HALF)], sws[b]))
        wdescs[-2].wait()
        wdescs[-1].wait()

    return gather_kernel(p0, p1, p2, p3, t0, t1, t2, t3)


def _mlp_body(e0_ref, e1_ref, e2_ref, e3_ref, oh_ref, w1_ref, b1_ref,
              w2_ref, b2_ref, o_ref):
    sel = jnp.transpose(oh_ref[...], (1, 0))
    h = b1_ref[...]
    e_refs = (e0_ref, e1_ref, e2_ref, e3_ref)
    for k in range(4):
        feat = jnp.zeros((e0_ref.shape[0], _ED), jnp.float32)
        for s in range(_PACK):
            m = sel[:, 4 * k + s : 4 * k + s + 1] > 0.5
            feat = feat + jnp.where(m, e_refs[k][:, _ED * s:_ED * (s + 1)], 0.0)
        h = h + jnp.dot(feat, w1_ref[_ED * k:_ED * (k + 1), :],
                        preferred_element_type=jnp.float32)
    h = jnp.maximum(h, 0.0)
    o_ref[...] = jnp.dot(h, w2_ref[...],
                         preferred_element_type=jnp.float32) + b2_ref[...]


def _mlp(e0, e1, e2, e3, oh, W1, b1, W2, b2):
    full = lambda i: (0, 0)
    espec = lambda: pl.BlockSpec((_MLP_BLOCK, _LANES), lambda i: (i, 0))
    return pl.pallas_call(
        _mlp_body,
        grid=(_BATCH // _MLP_BLOCK,),
        in_specs=[
            espec(), espec(), espec(), espec(),
            pl.BlockSpec((16, _MLP_BLOCK), lambda i: (0, i)),
            pl.BlockSpec((128, 64), full),
            pl.BlockSpec((1, 64), full),
            pl.BlockSpec((64, 32), full),
            pl.BlockSpec((1, 32), full),
        ],
        out_specs=pl.BlockSpec((_MLP_BLOCK, 32), lambda i: (i, 0)),
        out_shape=jax.ShapeDtypeStruct((_BATCH, 32), jnp.float32),
    )(e0, e1, e2, e3, oh, W1, b1.reshape(1, 64), W2, b2.reshape(1, 32))


def kernel(room_id, hotel, room_type, room_name,
           room_table, hotel_table, room_type_table, room_name_table,
           W1, b1, W2, b2):
    idxs = (room_id, hotel, room_type, room_name)
    vqs = (_VQ_BIG, _VQ_SMALL, _VQ_SMALL, _VQ_BIG)
    pb0, pb1 = _pack2(room_table, room_name_table, _VQ_BIG, _PBLK)
    ps0, ps1 = _pack2(hotel_table, room_type_table, _VQ_SMALL, _VQ_SMALL)
    packed = (pb0, ps0, ps1, pb1)
    s = tuple(i // vq for i, vq in zip(idxs, vqs))
    pidx = tuple(i - sk * vq for i, sk, vq in zip(idxs, s, vqs))
    sub = jnp.stack(s, axis=0)
    oh = (sub[:, None, :] == jnp.arange(_PACK, dtype=jnp.int32)[None, :, None])
    oh = oh.reshape(16, _BATCH).astype(jnp.float32)
    e0, e1, e2, e3 = _gather4(*pidx, *packed)
    return _mlp(e0, e1, e2, e3, oh, W1, b1, W2, b2)


# speedup vs baseline: 1.2518x; 1.2518x over previous
import functools

import jax
import jax.numpy as jnp
from jax import lax
from jax.experimental import pallas as pl
from jax.experimental.pallas import tpu as pltpu
from jax.experimental.pallas import tpu_sc as plsc

_BATCH = 16384
_ED = 32
_LANES = 128
_PACK = _LANES // _ED

_VQ_BIG = 25088
_VQ_SMALL = 256
_PBLK = 512

_NC, _NS = 2, 16
_NW = _NC * _NS
_BPW = _BATCH // _NW
_HALF = _BPW // 2
_CHUNK = 128

_MLP_BLOCK = 4096


def _pack_table(t, vq):
    v = t.shape[0]
    parts = []
    for s in range(_PACK):
        lo = s * vq
        hi = min((s + 1) * vq, v)
        q = t[lo:hi]
        if hi - lo < vq:
            q = jnp.pad(q, ((0, vq - (hi - lo)), (0, 0)))
        parts.append(q)
    return jnp.concatenate(parts, axis=1)


def _gather4(p0, p1, p2, p3, t0, t1, t2, t3):
    mesh = plsc.VectorSubcoreMesh(core_axis_name="c", subcore_axis_name="s")
    out = jax.ShapeDtypeStruct((_BATCH, _LANES), jnp.float32)
    fp = jnp.float32

    @functools.partial(
        pl.kernel, out_type=(out, out, out, out), mesh=mesh,
        scratch_types=[
            pltpu.VMEM((_BPW,), jnp.int32), pltpu.VMEM((_BPW,), jnp.int32),
            pltpu.VMEM((_BPW,), jnp.int32), pltpu.VMEM((_BPW,), jnp.int32),
            pltpu.VMEM((_HALF, _LANES), fp), pltpu.VMEM((_HALF, _LANES), fp),
            pltpu.SemaphoreType.DMA, pltpu.SemaphoreType.DMA,
            pltpu.SemaphoreType.DMA, pltpu.SemaphoreType.DMA,
        ])
    def gather_kernel(i0_hbm, i1_hbm, i2_hbm, i3_hbm,
                      t0_hbm, t1_hbm, t2_hbm, t3_hbm,
                      o0_hbm, o1_hbm, o2_hbm, o3_hbm,
                      iv0, iv1, iv2, iv3, rows0, rows1,
                      sg0, sg1, sw0, sw1):
        wid = lax.axis_index("s") * _NC + lax.axis_index("c")
        base = wid * _BPW
        i_hbms = (i0_hbm, i1_hbm, i2_hbm, i3_hbm)
        t_hbms = (t0_hbm, t1_hbm, t2_hbm, t3_hbm)
        o_hbms = (o0_hbm, o1_hbm, o2_hbm, o3_hbm)
        ivs = (iv0, iv1, iv2, iv3)
        rows = (rows0, rows1)
        sgs = (sg0, sg1)
        sws = (sw0, sw1)
        for k in range(4):
            pltpu.sync_copy(i_hbms[k].at[pl.ds(base, _BPW)], ivs[k])
        wdescs = []
        items = [(k, h) for k in range(4) for h in range(2)]
        for i, (k, h) in enumerate(items):
            b = i % 2
            if i >= 2:
                wdescs[i - 2].wait()
            gd = []
            for c in range(_HALF // _CHUNK):
                isl = pl.ds(h * _HALF + c * _CHUNK, _CHUNK)
                gd.append(pltpu.async_copy(
                    t_hbms[k].at[ivs[k].at[isl]],
                    rows[b].at[pl.ds(c * _CHUNK, _CHUNK)], sgs[b]))
            for d in gd:
                d.wait()
            wdescs.append(pltpu.async_copy(
                rows[b], o_hbms[k].at[pl.ds(base + h * _HALF, _HALF)], sws[b]))
        wdescs[-2].wait()
        wdescs[-1].wait()

    return gather_kernel(p0, p1, p2, p3, t0, t1, t2, t3)


def _mlp_body(e0_ref, e1_ref, e2_ref, e3_ref, oh_ref, w1_ref, b1_ref,
              w2_ref, b2_ref, o_ref):
    sel = jnp.transpose(oh_ref[...], (1, 0))
    h = b1_ref[...]
    e_refs = (e0_ref, e1_ref, e2_ref, e3_ref)
    for k in range(4):
        feat = jnp.zeros((e0_ref.shape[0], _ED), jnp.float32)
        for s in range(_PACK):
            m = sel[:, 4 * k + s : 4 * k + s + 1] > 0.5
            feat = feat + jnp.where(m, e_refs[k][:, _ED * s:_ED * (s + 1)], 0.0)
        h = h + jnp.dot(feat, w1_ref[_ED * k:_ED * (k + 1), :],
                        preferred_element_type=jnp.float32)
    h = jnp.maximum(h, 0.0)
    o_ref[...] = jnp.dot(h, w2_ref[...],
                         preferred_element_type=jnp.float32) + b2_ref[...]


def _mlp(e0, e1, e2, e3, oh, W1, b1, W2, b2):
    full = lambda i: (0, 0)
    espec = lambda: pl.BlockSpec((_MLP_BLOCK, _LANES), lambda i: (i, 0))
    return pl.pallas_call(
        _mlp_body,
        grid=(_BATCH // _MLP_BLOCK,),
        in_specs=[
            espec(), espec(), espec(), espec(),
            pl.BlockSpec((16, _MLP_BLOCK), lambda i: (0, i)),
            pl.BlockSpec((128, 64), full),
            pl.BlockSpec((1, 64), full),
            pl.BlockSpec((64, 32), full),
            pl.BlockSpec((1, 32), full),
        ],
        out_specs=pl.BlockSpec((_MLP_BLOCK, 32), lambda i: (i, 0)),
        out_shape=jax.ShapeDtypeStruct((_BATCH, 32), jnp.float32),
    )(e0, e1, e2, e3, oh, W1, b1.reshape(1, 64), W2, b2.reshape(1, 32))


def kernel(room_id, hotel, room_type, room_name,
           room_table, hotel_table, room_type_table, room_name_table,
           W1, b1, W2, b2):
    idxs = (room_id, hotel, room_type, room_name)
    vqs = (_VQ_BIG, _VQ_SMALL, _VQ_SMALL, _VQ_BIG)
    packed = tuple(_pack_table(t, vq)
                   for t, vq in zip((room_table, hotel_table, room_type_table,
                                     room_name_table), vqs))
    s = tuple(i // vq for i, vq in zip(idxs, vqs))
    pidx = tuple(i - sk * vq for i, sk, vq in zip(idxs, s, vqs))
    sub = jnp.stack(s, axis=0)
    oh = (sub[:, None, :] == jnp.arange(_PACK, dtype=jnp.int32)[None, :, None])
    oh = oh.reshape(16, _BATCH).astype(jnp.float32)
    e0, e1, e2, e3 = _gather4(*pidx, *packed)
    return _mlp(e0, e1, e2, e3, oh, W1, b1, W2, b2)


# speedup vs baseline: 2.1045x; 1.6812x over previous
import functools

import jax
import jax.numpy as jnp
from jax import lax
from jax.experimental import pallas as pl
from jax.experimental.pallas import tpu as pltpu
from jax.experimental.pallas import tpu_sc as plsc

_BATCH = 16384
_ED = 32
_LANES = 128

_NC, _NS = 2, 16
_NW = _NC * _NS
_BPW = _BATCH // _NW
_HALF = _BPW // 2
_CHUNK = 128

_MLP_BLOCK = 4096


def _gather4(i0, i1, i2, i3, t0, t1, t2, t3):
    mesh = plsc.VectorSubcoreMesh(core_axis_name="c", subcore_axis_name="s")
    out = jax.ShapeDtypeStruct((_BATCH, _LANES), jnp.float32)
    fp = jnp.float32

    @functools.partial(
        pl.kernel, out_type=(out, out, out, out), mesh=mesh,
        scratch_types=[
            pltpu.VMEM((_BPW,), jnp.int32), pltpu.VMEM((_BPW,), jnp.int32),
            pltpu.VMEM((_BPW,), jnp.int32), pltpu.VMEM((_BPW,), jnp.int32),
            pltpu.VMEM((_HALF, _LANES), fp), pltpu.VMEM((_HALF, _LANES), fp),
            pltpu.SemaphoreType.DMA, pltpu.SemaphoreType.DMA,
            pltpu.SemaphoreType.DMA, pltpu.SemaphoreType.DMA,
        ])
    def gather_kernel(i0_hbm, i1_hbm, i2_hbm, i3_hbm,
                      t0_hbm, t1_hbm, t2_hbm, t3_hbm,
                      o0_hbm, o1_hbm, o2_hbm, o3_hbm,
                      iv0, iv1, iv2, iv3, rows0, rows1,
                      sg0, sg1, sw0, sw1):
        wid = lax.axis_index("s") * _NC + lax.axis_index("c")
        base = wid * _BPW
        i_hbms = (i0_hbm, i1_hbm, i2_hbm, i3_hbm)
        t_hbms = (t0_hbm, t1_hbm, t2_hbm, t3_hbm)
        o_hbms = (o0_hbm, o1_hbm, o2_hbm, o3_hbm)
        ivs = (iv0, iv1, iv2, iv3)
        rows = (rows0, rows1)
        sgs = (sg0, sg1)
        sws = (sw0, sw1)
        for k in range(4):
            pltpu.sync_copy(i_hbms[k].at[pl.ds(base, _BPW)], ivs[k])
        wdescs = []
        items = [(k, h) for k in range(4) for h in range(2)]
        for i, (k, h) in enumerate(items):
            b = i % 2
            if i >= 2:
                wdescs[i - 2].wait()
            gd = []
            for c in range(_HALF // _CHUNK):
                isl = pl.ds(h * _HALF + c * _CHUNK, _CHUNK)
                gd.append(pltpu.async_copy(
                    t_hbms[k].at[ivs[k].at[isl]],
                    rows[b].at[pl.ds(c * _CHUNK, _CHUNK)], sgs[b]))
            for d in gd:
                d.wait()
            wdescs.append(pltpu.async_copy(
                rows[b], o_hbms[k].at[pl.ds(base + h * _HALF, _HALF)], sws[b]))
        wdescs[-2].wait()
        wdescs[-1].wait()

    return gather_kernel(i0, i1, i2, i3, t0, t1, t2, t3)


def _mlp_body(e0_ref, e1_ref, e2_ref, e3_ref, w1_ref, b1_ref,
              w2_ref, b2_ref, o_ref):
    h = b1_ref[...]
    for k, e in enumerate((e0_ref, e1_ref, e2_ref, e3_ref)):
        h = h + jnp.dot(e[:, 0:_ED], w1_ref[_ED * k:_ED * (k + 1), :],
                        preferred_element_type=jnp.float32)
    h = jnp.maximum(h, 0.0)
    o_ref[...] = jnp.dot(h, w2_ref[...],
                         preferred_element_type=jnp.float32) + b2_ref[...]


def _mlp(e0, e1, e2, e3, W1, b1, W2, b2):
    full = lambda i: (0, 0)
    espec = lambda: pl.BlockSpec((_MLP_BLOCK, _LANES), lambda i: (i, 0))
    return pl.pallas_call(
        _mlp_body,
        grid=(_BATCH // _MLP_BLOCK,),
        in_specs=[
            espec(), espec(), espec(), espec(),
            pl.BlockSpec((128, 64), full),
            pl.BlockSpec((1, 64), full),
            pl.BlockSpec((64, 32), full),
            pl.BlockSpec((1, 32), full),
        ],
        out_specs=pl.BlockSpec((_MLP_BLOCK, 32), lambda i: (i, 0)),
        out_shape=jax.ShapeDtypeStruct((_BATCH, 32), jnp.float32),
    )(e0, e1, e2, e3, W1, b1.reshape(1, 64), W2, b2.reshape(1, 32))


def kernel(room_id, hotel, room_type, room_name,
           room_table, hotel_table, room_type_table, room_name_table,
           W1, b1, W2, b2):
    pad = lambda t: jnp.pad(t, ((0, 0), (0, _LANES - _ED)))
    e0, e1, e2, e3 = _gather4(
        room_id, hotel, room_type, room_name,
        pad(room_table), pad(hotel_table), pad(room_type_table),
        pad(room_name_table))
    return _mlp(e0, e1, e2, e3, W1, b1, W2, b2)


# speedup vs baseline: 2.1838x; 1.0377x over previous
import functools

import jax
import jax.numpy as jnp
from jax import lax
from jax.experimental import pallas as pl
from jax.experimental.pallas import tpu as pltpu
from jax.experimental.pallas import tpu_sc as plsc

_BATCH = 16384
_ED = 32
_LANES = 128

_NC, _NS = 2, 16
_NW = _NC * _NS
_BPW = _BATCH // _NW
_HALF = _BPW // 2
_CHUNK = 128

_MLP_BLOCK = 4096


def _gather4(i0, i1, i2, i3, tbig, tsmall):
    mesh = plsc.VectorSubcoreMesh(core_axis_name="c", subcore_axis_name="s")
    out = jax.ShapeDtypeStruct((_BATCH, _LANES), jnp.float32)
    fp = jnp.float32

    @functools.partial(
        pl.kernel, out_type=(out, out, out, out), mesh=mesh,
        scratch_types=[
            pltpu.VMEM((_BPW,), jnp.int32), pltpu.VMEM((_BPW,), jnp.int32),
            pltpu.VMEM((_BPW,), jnp.int32), pltpu.VMEM((_BPW,), jnp.int32),
            pltpu.VMEM((_HALF, _LANES), fp), pltpu.VMEM((_HALF, _LANES), fp),
            pltpu.SemaphoreType.DMA, pltpu.SemaphoreType.DMA,
            pltpu.SemaphoreType.DMA, pltpu.SemaphoreType.DMA,
        ])
    def gather_kernel(i0_hbm, i1_hbm, i2_hbm, i3_hbm,
                      tb_hbm, ts_hbm,
                      o0_hbm, o1_hbm, o2_hbm, o3_hbm,
                      iv0, iv1, iv2, iv3, rows0, rows1,
                      sg0, sg1, sw0, sw1):
        wid = lax.axis_index("s") * _NC + lax.axis_index("c")
        base = wid * _BPW
        i_hbms = (i0_hbm, i1_hbm, i2_hbm, i3_hbm)
        t_hbms = (tb_hbm, ts_hbm, ts_hbm, tb_hbm)
        o_hbms = (o0_hbm, o1_hbm, o2_hbm, o3_hbm)
        ivs = (iv0, iv1, iv2, iv3)
        rows = (rows0, rows1)
        sgs = (sg0, sg1)
        sws = (sw0, sw1)
        for k in range(4):
            pltpu.sync_copy(i_hbms[k].at[pl.ds(base, _BPW)], ivs[k])
        wdescs = []
        items = [(k, h) for k in range(4) for h in range(2)]
        for i, (k, h) in enumerate(items):
            b = i % 2
            if i >= 2:
                wdescs[i - 2].wait()
            gd = []
            for c in range(_HALF // _CHUNK):
                isl = pl.ds(h * _HALF + c * _CHUNK, _CHUNK)
                gd.append(pltpu.async_copy(
                    t_hbms[k].at[ivs[k].at[isl]],
                    rows[b].at[pl.ds(c * _CHUNK, _CHUNK)], sgs[b]))
            for d in gd:
                d.wait()
            wdescs.append(pltpu.async_copy(
                rows[b], o_hbms[k].at[pl.ds(base + h * _HALF, _HALF)], sws[b]))
        wdescs[-2].wait()
        wdescs[-1].wait()

    return gather_kernel(i0, i1, i2, i3, tbig, tsmall)


def _mlp_body(e0_ref, e1_ref, e2_ref, e3_ref, w1_ref, b1_ref,
              w2_ref, b2_ref, o_ref):
    h = b1_ref[...]
    offs = (0, 0, _ED, _ED)
    for k, e in enumerate((e0_ref, e1_ref, e2_ref, e3_ref)):
        h = h + jnp.dot(e[:, offs[k]:offs[k] + _ED],
                        w1_ref[_ED * k:_ED * (k + 1), :],
                        preferred_element_type=jnp.float32)
    h = jnp.maximum(h, 0.0)
    o_ref[...] = jnp.dot(h, w2_ref[...],
                         preferred_element_type=jnp.float32) + b2_ref[...]


def _mlp(e0, e1, e2, e3, W1, b1, W2, b2):
    full = lambda i: (0, 0)
    espec = lambda: pl.BlockSpec((_MLP_BLOCK, _LANES), lambda i: (i, 0))
    return pl.pallas_call(
        _mlp_body,
        grid=(_BATCH // _MLP_BLOCK,),
        in_specs=[
            espec(), espec(), espec(), espec(),
            pl.BlockSpec((128, 64), full),
            pl.BlockSpec((1, 64), full),
            pl.BlockSpec((64, 32), full),
            pl.BlockSpec((1, 32), full),
        ],
        out_specs=pl.BlockSpec((_MLP_BLOCK, 32), lambda i: (i, 0)),
        out_shape=jax.ShapeDtypeStruct((_BATCH, 32), jnp.float32),
    )(e0, e1, e2, e3, W1, b1.reshape(1, 64), W2, b2.reshape(1, 32))


def kernel(room_id, hotel, room_type, room_name,
           room_table, hotel_table, room_type_table, room_name_table,
           W1, b1, W2, b2):
    def combo(ta, tb):
        z = jnp.zeros((ta.shape[0], _LANES - 2 * _ED), jnp.float32)
        return jnp.concatenate([ta, tb, z], axis=1)

    e0, e1, e2, e3 = _gather4(
        room_id, hotel, room_type, room_name,
        combo(room_table, room_name_table),
        combo(hotel_table, room_type_table))
    return _mlp(e0, e1, e2, e3, W1, b1, W2, b2)


# speedup vs baseline: 2.1880x; 1.0019x over previous
import functools

import jax
import jax.numpy as jnp
from jax import lax
from jax.experimental import pallas as pl
from jax.experimental.pallas import tpu as pltpu
from jax.experimental.pallas import tpu_sc as plsc

_BATCH = 16384
_ED = 32
_LANES = 128

_NC, _NS = 2, 16
_NW = _NC * _NS
_BPW = _BATCH // _NW
_HALF = _BPW // 2
_CHUNK = 128

_MLP_BLOCK = 4096


def _gather4(i0, i1, i2, i3, tbig, tsmall):
    mesh = plsc.VectorSubcoreMesh(core_axis_name="c", subcore_axis_name="s")
    out = jax.ShapeDtypeStruct((_BATCH, _LANES), jnp.float32)
    fp = jnp.float32

    @functools.partial(
        pl.kernel, out_type=(out, out, out, out), mesh=mesh,
        scratch_types=[
            pltpu.VMEM((_BPW,), jnp.int32), pltpu.VMEM((_BPW,), jnp.int32),
            pltpu.VMEM((_BPW,), jnp.int32), pltpu.VMEM((_BPW,), jnp.int32),
            pltpu.VMEM((_HALF, _LANES), fp), pltpu.VMEM((_HALF, _LANES), fp),
            pltpu.SemaphoreType.DMA, pltpu.SemaphoreType.DMA,
            pltpu.SemaphoreType.DMA, pltpu.SemaphoreType.DMA,
        ])
    def gather_kernel(i0_hbm, i1_hbm, i2_hbm, i3_hbm,
                      tb_hbm, ts_hbm,
                      o0_hbm, o1_hbm, o2_hbm, o3_hbm,
                      iv0, iv1, iv2, iv3, rows0, rows1,
                      sg0, sg1, sw0, sw1):
        wid = lax.axis_index("s") * _NC + lax.axis_index("c")
        base = wid * _BPW
        i_hbms = (i0_hbm, i1_hbm, i2_hbm, i3_hbm)
        t_hbms = (tb_hbm, ts_hbm, ts_hbm, tb_hbm)
        o_hbms = (o0_hbm, o1_hbm, o2_hbm, o3_hbm)
        ivs = (iv0, iv1, iv2, iv3)
        rows = (rows0, rows1)
        sgs = (sg0, sg1)
        sws = (sw0, sw1)
        for k in range(4):
            pltpu.sync_copy(i_hbms[k].at[pl.ds(base, _BPW)], ivs[k])
        wdescs = []
        items = [(k, h) for k in range(4) for h in range(2)]
        for i, (k, h) in enumerate(items):
            b = i % 2
            if i >= 2:
                wdescs[i - 2].wait()
            gd = []
            for c in range(_HALF // _CHUNK):
                isl = pl.ds(h * _HALF + c * _CHUNK, _CHUNK)
                gd.append(pltpu.async_copy(
                    t_hbms[k].at[ivs[k].at[isl]],
                    rows[b].at[pl.ds(c * _CHUNK, _CHUNK)], sgs[b]))
            for d in gd:
                d.wait()
            wdescs.append(pltpu.async_copy(
                rows[b], o_hbms[k].at[pl.ds(base + h * _HALF, _HALF)], sws[b]))
        wdescs[-2].wait()
        wdescs[-1].wait()

    return gather_kernel(i0, i1, i2, i3, tbig, tsmall)


def _mlp_body(e0_ref, e1_ref, e2_ref, e3_ref, w1_ref, b1_ref,
              w2_ref, b2_ref, o_ref):
    h = b1_ref[...]
    offs = (0, 0, _ED, _ED)
    for k, e in enumerate((e0_ref, e1_ref, e2_ref, e3_ref)):
        h = h + jnp.dot(e[:, offs[k]:offs[k] + _ED],
                        w1_ref[_ED * k:_ED * (k + 1), :],
                        preferred_element_type=jnp.float32)
    h = jnp.maximum(h, 0.0)
    o_ref[...] = jnp.dot(h, w2_ref[...],
                         preferred_element_type=jnp.float32) + b2_ref[...]


def _mlp(e0, e1, e2, e3, W1, b1, W2, b2):
    full = lambda i: (0, 0)
    espec = lambda: pl.BlockSpec((_MLP_BLOCK, _LANES), lambda i: (i, 0))
    return pl.pallas_call(
        _mlp_body,
        grid=(_BATCH // _MLP_BLOCK,),
        in_specs=[
            espec(), espec(), espec(), espec(),
            pl.BlockSpec((128, 64), full),
            pl.BlockSpec((1, 64), full),
            pl.BlockSpec((64, 32), full),
            pl.BlockSpec((1, 32), full),
        ],
        out_specs=pl.BlockSpec((_MLP_BLOCK, 32), lambda i: (i, 0)),
        out_shape=jax.ShapeDtypeStruct((_BATCH, 32), jnp.float32),
    )(e0, e1, e2, e3, W1, b1.reshape(1, 64), W2, b2.reshape(1, 32))


def kernel(room_id, hotel, room_type, room_name,
           room_table, hotel_table, room_type_table, room_name_table,
           W1, b1, W2, b2):
    def combo(ta, tb):
        return jnp.pad(jnp.concatenate([ta, tb], axis=1),
                       ((0, 0), (0, _LANES - 2 * _ED)))

    e0, e1, e2, e3 = _gather4(
        room_id, hotel, room_type, room_name,
        combo(room_table, room_name_table),
        combo(hotel_table, room_type_table))
    return _mlp(e0, e1, e2, e3, W1, b1, W2, b2)


# speedup vs baseline: 2.1943x; 1.0029x over previous
import functools

import jax
import jax.numpy as jnp
from jax import lax
from jax.experimental import pallas as pl
from jax.experimental.pallas import tpu as pltpu
from jax.experimental.pallas import tpu_sc as plsc

_BATCH = 16384
_ED = 32
_LANES = 128

_NC, _NS = 2, 16
_NW = _NC * _NS
_BPW = _BATCH // _NW
_HALF = _BPW // 2
_CHUNK = 128

_MLP_BLOCK = 4096


def _gather2(ia, ib, table):
    mesh = plsc.VectorSubcoreMesh(core_axis_name="c", subcore_axis_name="s")
    out = jax.ShapeDtypeStruct((_BATCH, _LANES), jnp.float32)
    fp = jnp.float32

    @functools.partial(
        pl.kernel, out_type=(out, out), mesh=mesh,
        scratch_types=[
            pltpu.VMEM((_BPW,), jnp.int32), pltpu.VMEM((_BPW,), jnp.int32),
            pltpu.VMEM((_HALF, _LANES), fp), pltpu.VMEM((_HALF, _LANES), fp),
            pltpu.SemaphoreType.DMA, pltpu.SemaphoreType.DMA,
            pltpu.SemaphoreType.DMA, pltpu.SemaphoreType.DMA,
        ])
    def gather_kernel(ia_hbm, ib_hbm, t_hbm, oa_hbm, ob_hbm,
                      iva, ivb, rows0, rows1, sg0, sg1, sw0, sw1):
        wid = lax.axis_index("s") * _NC + lax.axis_index("c")
        base = wid * _BPW
        i_hbms = (ia_hbm, ib_hbm)
        o_hbms = (oa_hbm, ob_hbm)
        ivs = (iva, ivb)
        rows = (rows0, rows1)
        sgs = (sg0, sg1)
        sws = (sw0, sw1)
        for k in range(2):
            pltpu.sync_copy(i_hbms[k].at[pl.ds(base, _BPW)], ivs[k])
        wdescs = []
        items = [(k, h) for k in range(2) for h in range(2)]
        for i, (k, h) in enumerate(items):
            b = i % 2
            if i >= 2:
                wdescs[i - 2].wait()
            gd = []
            for c in range(_HALF // _CHUNK):
                isl = pl.ds(h * _HALF + c * _CHUNK, _CHUNK)
                gd.append(pltpu.async_copy(
                    t_hbm.at[ivs[k].at[isl]],
                    rows[b].at[pl.ds(c * _CHUNK, _CHUNK)], sgs[b]))
            for d in gd:
                d.wait()
            wdescs.append(pltpu.async_copy(
                rows[b], o_hbms[k].at[pl.ds(base + h * _HALF, _HALF)], sws[b]))
        wdescs[-2].wait()
        wdescs[-1].wait()

    return gather_kernel(ia, ib, table)


def _mlp_partial_body(e1_ref, e2_ref, w1_ref, b1_ref, h_ref):
    h = b1_ref[...]
    h = h + jnp.dot(e1_ref[:, 0:_ED], w1_ref[_ED:2 * _ED, :],
                    preferred_element_type=jnp.float32)
    h = h + jnp.dot(e2_ref[:, _ED:2 * _ED], w1_ref[2 * _ED:3 * _ED, :],
                    preferred_element_type=jnp.float32)
    h_ref[...] = h


def _mlp_final_body(h_ref, e0_ref, e3_ref, w1_ref, w2_ref, b2_ref, o_ref):
    h = h_ref[...]
    h = h + jnp.dot(e0_ref[:, 0:_ED], w1_ref[0:_ED, :],
                    preferred_element_type=jnp.float32)
    h = h + jnp.dot(e3_ref[:, _ED:2 * _ED], w1_ref[3 * _ED:4 * _ED, :],
                    preferred_element_type=jnp.float32)
    h = jnp.maximum(h, 0.0)
    o_ref[...] = jnp.dot(h, w2_ref[...],
                         preferred_element_type=jnp.float32) + b2_ref[...]


def _mlp(es1, es2, eb0, eb3, W1, b1, W2, b2):
    full = lambda i: (0, 0)
    espec = lambda: pl.BlockSpec((_MLP_BLOCK, _LANES), lambda i: (i, 0))
    hspec = pl.BlockSpec((_MLP_BLOCK, 64), lambda i: (i, 0))
    grid = (_BATCH // _MLP_BLOCK,)
    h0 = pl.pallas_call(
        _mlp_partial_body,
        grid=grid,
        in_specs=[espec(), espec(),
                  pl.BlockSpec((128, 64), full),
                  pl.BlockSpec((1, 64), full)],
        out_specs=hspec,
        out_shape=jax.ShapeDtypeStruct((_BATCH, 64), jnp.float32),
    )(es1, es2, W1, b1.reshape(1, 64))
    return pl.pallas_call(
        _mlp_final_body,
        grid=grid,
        in_specs=[hspec, espec(), espec(),
                  pl.BlockSpec((128, 64), full),
                  pl.BlockSpec((64, 32), full),
                  pl.BlockSpec((1, 32), full)],
        out_specs=pl.BlockSpec((_MLP_BLOCK, 32), lambda i: (i, 0)),
        out_shape=jax.ShapeDtypeStruct((_BATCH, 32), jnp.float32),
    )(h0, eb0, eb3, W1, W2, b2.reshape(1, 32))


def kernel(room_id, hotel, room_type, room_name,
           room_table, hotel_table, room_type_table, room_name_table,
           W1, b1, W2, b2):
    def combo(ta, tb):
        return jnp.pad(jnp.concatenate([ta, tb], axis=1),
                       ((0, 0), (0, _LANES - 2 * _ED)))

    es1, es2 = _gather2(hotel, room_type, combo(hotel_table, room_type_table))
    eb0, eb3 = _gather2(room_id, room_name, combo(room_table, room_name_table))
    return _mlp(es1, es2, eb0, eb3, W1, b1, W2, b2)


# speedup vs baseline: 2.2021x; 1.0035x over previous
import functools

import jax
import jax.numpy as jnp
from jax import lax
from jax.experimental import pallas as pl
from jax.experimental.pallas import tpu as pltpu
from jax.experimental.pallas import tpu_sc as plsc

_BATCH = 16384
_ED = 32
_LANES = 128

_NC, _NS = 2, 16
_NW = _NC * _NS
_BPW = _BATCH // _NW
_HALF = _BPW // 2
_CHUNK = 128

_MLP_BLOCK = 4096


def _gather2(ia, ib, table):
    mesh = plsc.VectorSubcoreMesh(core_axis_name="c", subcore_axis_name="s")
    out = jax.ShapeDtypeStruct((_BATCH, _LANES), jnp.float32)
    fp = jnp.float32

    @functools.partial(
        pl.kernel, out_type=(out, out), mesh=mesh,
        scratch_types=[
            pltpu.VMEM((_BPW,), jnp.int32), pltpu.VMEM((_BPW,), jnp.int32),
            pltpu.VMEM((_HALF, _LANES), fp), pltpu.VMEM((_HALF, _LANES), fp),
            pltpu.SemaphoreType.DMA, pltpu.SemaphoreType.DMA,
            pltpu.SemaphoreType.DMA, pltpu.SemaphoreType.DMA,
        ])
    def gather_kernel(ia_hbm, ib_hbm, t_hbm, oa_hbm, ob_hbm,
                      iva, ivb, rows0, rows1, sg0, sg1, sw0, sw1):
        wid = lax.axis_index("s") * _NC + lax.axis_index("c")
        base = wid * _BPW
        i_hbms = (ia_hbm, ib_hbm)
        o_hbms = (oa_hbm, ob_hbm)
        ivs = (iva, ivb)
        rows = (rows0, rows1)
        sgs = (sg0, sg1)
        sws = (sw0, sw1)
        for k in range(2):
            pltpu.sync_copy(i_hbms[k].at[pl.ds(base, _BPW)], ivs[k])
        wdescs = []
        items = [(k, h) for k in range(2) for h in range(2)]
        for i, (k, h) in enumerate(items):
            b = i % 2
            if i >= 2:
                wdescs[i - 2].wait()
            gd = []
            for c in range(_HALF // _CHUNK):
                isl = pl.ds(h * _HALF + c * _CHUNK, _CHUNK)
                gd.append(pltpu.async_copy(
                    t_hbm.at[ivs[k].at[isl]],
                    rows[b].at[pl.ds(c * _CHUNK, _CHUNK)], sgs[b]))
            for d in gd:
                d.wait()
            wdescs.append(pltpu.async_copy(
                rows[b], o_hbms[k].at[pl.ds(base + h * _HALF, _HALF)], sws[b]))
        wdescs[-2].wait()
        wdescs[-1].wait()

    return gather_kernel(ia, ib, table)


def _mlp_partial_body(e1_ref, e2_ref, w1_ref, b1_ref, h_ref):
    h = b1_ref[...]
    h = h + jnp.dot(e1_ref[:, 0:_ED], w1_ref[_ED:2 * _ED, :],
                    preferred_element_type=jnp.float32)
    h = h + jnp.dot(e2_ref[:, _ED:2 * _ED], w1_ref[2 * _ED:3 * _ED, :],
                    preferred_element_type=jnp.float32)
    h_ref[...] = h


def _mlp_final_body(h_ref, e0_ref, e3_ref, w1_ref, w2_ref, b2_ref, o_ref):
    h = h_ref[...]
    h = h + jnp.dot(e0_ref[:, 0:_ED], w1_ref[0:_ED, :],
                    preferred_element_type=jnp.float32)
    h = h + jnp.dot(e3_ref[:, _ED:2 * _ED], w1_ref[3 * _ED:4 * _ED, :],
                    preferred_element_type=jnp.float32)
    h = jnp.maximum(h, 0.0)
    o_ref[...] = jnp.dot(h, w2_ref[...],
                         preferred_element_type=jnp.float32) + b2_ref[...]


def _mlp(es1, es2, eb0, eb3, W1, b1, W2, b2):
    full = lambda i: (0, 0)
    espec = lambda: pl.BlockSpec((_MLP_BLOCK, _LANES), lambda i: (i, 0))
    hspec = pl.BlockSpec((_MLP_BLOCK, 64), lambda i: (i, 0))
    grid = (_BATCH // _MLP_BLOCK,)
    h0 = pl.pallas_call(
        _mlp_partial_body,
        grid=grid,
        in_specs=[espec(), espec(),
                  pl.BlockSpec((128, 64), full),
                  pl.BlockSpec((1, 64), full)],
        out_specs=hspec,
        out_shape=jax.ShapeDtypeStruct((_BATCH, 64), jnp.float32),
    )(es1, es2, W1, b1.reshape(1, 64))
    return pl.pallas_call(
        _mlp_final_body,
        grid=grid,
        in_specs=[hspec, espec(), espec(),
                  pl.BlockSpec((128, 64), full),
                  pl.BlockSpec((64, 32), full),
                  pl.BlockSpec((1, 32), full)],
        out_specs=pl.BlockSpec((_MLP_BLOCK, 32), lambda i: (i, 0)),
        out_shape=jax.ShapeDtypeStruct((_BATCH, 32), jnp.float32),
    )(h0, eb0, eb3, W1, W2, b2.reshape(1, 32))


def kernel(room_id, hotel, room_type, room_name,
           room_table, hotel_table, room_type_table, room_name_table,
           W1, b1, W2, b2):
    def combo(ta, tb):
        z = jnp.zeros((ta.shape[0], _LANES - 2 * _ED), jnp.float32)
        return jnp.concatenate([ta, tb, z], axis=1)

    es1, es2 = _gather2(hotel, room_type, combo(hotel_table, room_type_table))
    eb0, eb3 = _gather2(room_id, room_name, combo(room_table, room_name_table))
    return _mlp(es1, es2, eb0, eb3, W1, b1, W2, b2)
